# Initial kernel scaffold; baseline (speedup 1.0000x reference)
import functools

import jax
import jax.numpy as jnp
from jax import lax
from jax.experimental import pallas as pl
from jax.experimental.pallas import tpu as pltpu
from jax.experimental.pallas import tpu_sc as plsc

NC, NS, L = 2, 16, 16
NW = NC * NS


def _sqrt16(v):
    i = plsc.bitcast(v, jnp.int32)
    y = plsc.bitcast(jnp.int32(0x5F3759DF) - (i >> 1), jnp.float32)
    for _ in range(3):
        y = y * (1.5 - 0.5 * v * y * y)
    return v * y


def _make_sc_kernel(B, V, D, R):
    H = D // 2
    BW = B // NW
    C = 128
    NCHUNK = BW // C

    mesh = plsc.VectorSubcoreMesh(core_axis_name="c", subcore_axis_name="s")

    @functools.partial(
        pl.kernel,
        out_type=jax.ShapeDtypeStruct((B,), jnp.float32),
        mesh=mesh,
        scratch_types=[
            pltpu.VMEM((C,), jnp.int32),
            pltpu.VMEM((C,), jnp.int32),
            pltpu.VMEM((C,), jnp.int32),
            pltpu.VMEM((C, D), jnp.float32),
            pltpu.VMEM((C, D), jnp.float32),
            pltpu.VMEM((C, H), jnp.float32),
            pltpu.VMEM((BW,), jnp.float32),
            pltpu.SemaphoreType.DMA,
            pltpu.SemaphoreType.DMA,
            pltpu.SemaphoreType.DMA,
        ],
    )
    def sc_kernel(hidx_hbm, tidx_hbm, ridx_hbm, ent_hbm, rel_hbm, out_hbm,
                  hidx_v, tidx_v, ridx_v, hbuf, tbuf, rbuf, score_v,
                  sem_h, sem_t, sem_r):
        wid = lax.axis_index("s") * NC + lax.axis_index("c")
        base = wid * BW

        for ci in range(NCHUNK):
            c0 = ci * C
            pltpu.sync_copy(hidx_hbm.at[pl.ds(base + c0, C)], hidx_v)
            pltpu.sync_copy(tidx_hbm.at[pl.ds(base + c0, C)], tidx_v)
            pltpu.sync_copy(ridx_hbm.at[pl.ds(base + c0, C)], ridx_v)
            cp_h = pltpu.async_copy(ent_hbm.at[hidx_v], hbuf, sem_h)
            cp_t = pltpu.async_copy(ent_hbm.at[tidx_v], tbuf, sem_t)
            cp_r = pltpu.async_copy(rel_hbm.at[ridx_v], rbuf, sem_r)
            cp_h.wait()
            cp_t.wait()
            cp_r.wait()

            def elem_body(e, carry):
                acc = jnp.zeros((L,), jnp.float32)
                for q in range(H // L):
                    x = rbuf[e, pl.ds(q * L, L)]
                    x2 = x * x
                    cosv = 1.0 - x2 * (0.5 - x2 * (1.0 / 24 - x2 * (1.0 / 720)))
                    sinv = x * (1.0 - x2 * (1.0 / 6 - x2 * (1.0 / 120 - x2 * (1.0 / 5040))))
                    hre = hbuf[e, pl.ds(q * L, L)]
                    him = hbuf[e, pl.ds(H + q * L, L)]
                    tre = tbuf[e, pl.ds(q * L, L)]
                    tim = tbuf[e, pl.ds(H + q * L, L)]
                    dre = hre * cosv - him * sinv - tre
                    dim_ = hre * sinv + him * cosv - tim
                    acc = acc + dre * dre + dim_ * dim_
                score_v[c0 + e] = jnp.sum(acc)
                return carry

            lax.fori_loop(0, C, elem_body, 0)

        def sqrt_body(g, carry):
            v = score_v[pl.ds(g * L, L)]
            score_v[pl.ds(g * L, L)] = _sqrt16(v)
            return carry

        lax.fori_loop(0, BW // L, sqrt_body, 0)
        pltpu.sync_copy(score_v, out_hbm.at[pl.ds(base, BW)])

    return sc_kernel


@jax.jit
def kernel(h_idx, r_idx, t_idx, entity_emb, rel_emb):
    B = h_idx.shape[0]
    V, D = entity_emb.shape
    R, H = rel_emb.shape
    sc = _make_sc_kernel(B, V, D, R)
    return sc(h_idx.astype(jnp.int32), t_idx.astype(jnp.int32),
              r_idx.astype(jnp.int32), entity_emb, rel_emb)


# speedup vs baseline: 1.1890x; 1.1890x over previous
import functools

import jax
import jax.numpy as jnp
from jax import lax
from jax.experimental import pallas as pl
from jax.experimental.pallas import tpu as pltpu
from jax.experimental.pallas import tpu_sc as plsc

NC, NS, L = 2, 16, 16
NW = NC * NS


def _sqrt16(v):
    i = plsc.bitcast(v, jnp.int32)
    y = plsc.bitcast(jnp.int32(0x5F3759DF) - (i >> 1), jnp.float32)
    for _ in range(3):
        y = y * (1.5 - 0.5 * v * y * y)
    return v * y


def _make_sc_kernel(B, V, D, R):
    H = D // 2
    BW = B // NW
    C = 128
    NCHUNK = BW // C

    mesh = plsc.VectorSubcoreMesh(core_axis_name="c", subcore_axis_name="s")

    @functools.partial(
        pl.kernel,
        out_type=jax.ShapeDtypeStruct((B,), jnp.float32),
        mesh=mesh,
        compiler_params=pltpu.CompilerParams(needs_layout_passes=False),
        scratch_types=[
            pltpu.VMEM((C,), jnp.int32),
            pltpu.VMEM((C,), jnp.int32),
            pltpu.VMEM((C,), jnp.int32),
            pltpu.VMEM((C, D), jnp.float32),
            pltpu.VMEM((C, D), jnp.float32),
            pltpu.VMEM((C, D), jnp.float32),
            pltpu.VMEM((BW,), jnp.float32),
            pltpu.SemaphoreType.DMA,
            pltpu.SemaphoreType.DMA,
            pltpu.SemaphoreType.DMA,
        ],
    )
    def sc_kernel(hidx_hbm, tidx_hbm, ridx_hbm, ent_hbm, rel_hbm, out_hbm,
                  hidx_v, tidx_v, ridx_v, hbuf, tbuf, rbuf, score_v,
                  sem_h, sem_t, sem_r):
        wid = lax.axis_index("s") * NC + lax.axis_index("c")
        base = wid * BW
        lane = lax.iota(jnp.int32, L)

        def chunk_body(ci, carry):
            c0 = ci * C
            pltpu.sync_copy(hidx_hbm.at[pl.ds(base + c0, C)], hidx_v)
            pltpu.sync_copy(tidx_hbm.at[pl.ds(base + c0, C)], tidx_v)
            pltpu.sync_copy(ridx_hbm.at[pl.ds(base + c0, C)], ridx_v)
            cp_h = pltpu.async_copy(ent_hbm.at[hidx_v], hbuf, sem_h)
            cp_t = pltpu.async_copy(ent_hbm.at[tidx_v], tbuf, sem_t)
            cp_r = pltpu.async_copy(rel_hbm.at[ridx_v], rbuf, sem_r)
            cp_h.wait()
            cp_t.wait()
            cp_r.wait()

            def group_body(g, carry2):
                ids = g * L + lane

                def d_body(d, acc):
                    dv = jnp.full((L,), d, jnp.int32)
                    x = plsc.load_gather(rbuf, [ids, dv])
                    x2 = x * x
                    cosv = 1.0 - x2 * (0.5 - x2 * (1.0 / 24 - x2 * (1.0 / 720)))
                    sinv = x * (1.0 - x2 * (1.0 / 6 - x2 * (1.0 / 120 - x2 * (1.0 / 5040))))
                    hre = plsc.load_gather(hbuf, [ids, dv])
                    him = plsc.load_gather(hbuf, [ids, dv + H])
                    tre = plsc.load_gather(tbuf, [ids, dv])
                    tim = plsc.load_gather(tbuf, [ids, dv + H])
                    dre = hre * cosv - him * sinv - tre
                    dim_ = hre * sinv + him * cosv - tim
                    return acc + dre * dre + dim_ * dim_

                acc = lax.fori_loop(0, H, d_body, jnp.zeros((L,), jnp.float32))
                score_v[pl.ds(c0 + g * L, L)] = _sqrt16(acc)
                return carry2

            lax.fori_loop(0, C // L, group_body, 0)
            return carry

        lax.fori_loop(0, NCHUNK, chunk_body, 0)
        pltpu.sync_copy(score_v, out_hbm.at[pl.ds(base, BW)])

    return sc_kernel


@jax.jit
def kernel(h_idx, r_idx, t_idx, entity_emb, rel_emb):
    B = h_idx.shape[0]
    V, D = entity_emb.shape
    R, H = rel_emb.shape
    sc = _make_sc_kernel(B, V, D, R)
    rel_pad = jnp.pad(rel_emb, ((0, 0), (0, D - H)))
    return sc(h_idx.astype(jnp.int32), t_idx.astype(jnp.int32),
              r_idx.astype(jnp.int32), entity_emb, rel_pad)
